# baseline (speedup 1.0000x reference)
import functools

import jax
import jax.numpy as jnp
from jax.experimental import pallas as pl

D = 512
TILE = 512

_NODE_SIZES = (17, 7, 2, 8, 2, 4, 7)
_EDGE_SIZES = (6, 2, 2, 6)


def _offsets(sizes):
    offs, t = [], 0
    for s in sizes:
        offs.append(t)
        t += s
    return tuple(offs), t


_NODE_OFFS, _NODE_TOT = _offsets(_NODE_SIZES)
_EDGE_OFFS, _EDGE_TOT = _offsets(_EDGE_SIZES)
_NODE_COLS = 48
_EDGE_COLS = 16


def _onehot_sum(idx, offs, ncols):
    rows = idx.shape[0]
    cols = jax.lax.broadcasted_iota(jnp.int32, (rows, ncols), 1)
    acc = None
    for k, off in enumerate(offs):
        term = (idx[:, k:k + 1] + off == cols).astype(jnp.float32)
        acc = term if acc is None else acc + term
    return acc


def _layer_norm(x, gamma, beta, eps=1e-5):
    m = jnp.mean(x, axis=-1, keepdims=True)
    v = jnp.mean((x - m) ** 2, axis=-1, keepdims=True)
    return (x - m) * jax.lax.rsqrt(v + eps) * gamma + beta


def _fused_kernel(nf_ref, ef_ref, tn_ref, te_ref, wc_ref, bs_ref, be_ref,
                  gn_ref, bn_ref, ge_ref, beta_e_ref,
                  nx_ref, ns_ref, ne_ref, ex_ref):
    n_oh = _onehot_sum(nf_ref[...], _NODE_OFFS, _NODE_COLS)
    nsum = jnp.dot(n_oh, tn_ref[...], preferred_element_type=jnp.float32)
    node_x = _layer_norm(nsum, gn_ref[...], bn_ref[...])
    nx_ref[...] = node_x
    proj = jax.lax.dot_general(
        node_x, wc_ref[...], (((1,), (1,)), ((), ())),
        preferred_element_type=jnp.float32)
    ns_ref[...] = proj[:, :D] + bs_ref[...]
    ne_ref[...] = proj[:, D:] + be_ref[...]
    e_oh = _onehot_sum(ef_ref[...], _EDGE_OFFS, _EDGE_COLS)
    esum = jnp.dot(e_oh, te_ref[...], preferred_element_type=jnp.float32)
    ex_ref[...] = _layer_norm(esum, ge_ref[...], beta_e_ref[...])


@jax.jit
def _run(nf, ef, tn, te, wc, bs, be, gn, bn, ge, bt):
    rows = nf.shape[0]
    grid = (rows // TILE,)

    def row_spec(width):
        return pl.BlockSpec((TILE, width), lambda i: (i, 0))

    def const_spec(a):
        return pl.BlockSpec(a.shape, lambda i: (0,) * a.ndim)

    out_shape = [jax.ShapeDtypeStruct((rows, D), jnp.float32) for _ in range(4)]
    return pl.pallas_call(
        _fused_kernel,
        grid=grid,
        in_specs=[
            row_spec(nf.shape[1]), row_spec(ef.shape[1]),
            const_spec(tn), const_spec(te),
            const_spec(wc), const_spec(bs), const_spec(be),
            const_spec(gn), const_spec(bn), const_spec(ge), const_spec(bt),
        ],
        out_specs=[row_spec(D) for _ in range(4)],
        out_shape=out_shape,
    )(nf, ef, tn, te, wc, bs, be, gn, bn, ge, bt)


def kernel(node_features, edge_features, emb_element, emb_degree, emb_ring,
           emb_hybrid, emb_aromatic, emb_chirality, emb_charge, W_start,
           b_start, W_end, b_end, emb_etype, emb_conj, emb_ering, emb_stereo,
           gamma_node, beta_node, gamma_edge, beta_edge):
    B, N, _ = node_features.shape
    rows = B * N
    nf = node_features.reshape(rows, -1).astype(jnp.int32)
    ef = edge_features.reshape(rows, -1).astype(jnp.int32)
    tn = jnp.concatenate([emb_element, emb_degree, emb_ring, emb_hybrid,
                          emb_aromatic, emb_chirality, emb_charge], axis=0)
    tn = jnp.pad(tn, ((0, _NODE_COLS - _NODE_TOT), (0, 0)))
    te = jnp.concatenate([emb_etype, emb_conj, emb_ering, emb_stereo], axis=0)
    wc = jnp.concatenate([W_start, W_end], axis=0)
    outs = _run(nf, ef, tn, te,
                wc, b_start.reshape(1, D), b_end.reshape(1, D),
                gamma_node.reshape(1, D), beta_node.reshape(1, D),
                gamma_edge.reshape(1, D), beta_edge.reshape(1, D))
    node_x, node_x_start, node_x_end, edge_x = [o.reshape(B, N, D) for o in outs]
    return (node_x, node_x_start, node_x_end, edge_x)


# speedup vs baseline: 1.1178x; 1.1178x over previous
import functools

import jax
import jax.numpy as jnp
from jax.experimental import pallas as pl

D = 512
TILE = 1024

_NODE_SIZES = (17, 7, 2, 8, 2, 4, 7)
_EDGE_SIZES = (6, 2, 2, 6)


def _offsets(sizes):
    offs, t = [], 0
    for s in sizes:
        offs.append(t)
        t += s
    return tuple(offs), t


_NODE_OFFS, _NODE_TOT = _offsets(_NODE_SIZES)
_EDGE_OFFS, _EDGE_TOT = _offsets(_EDGE_SIZES)
_NODE_COLS = 48
_EDGE_COLS = 16


def _onehot_sum(idx, offs, ncols):
    rows = idx.shape[0]
    cols = jax.lax.broadcasted_iota(jnp.int32, (rows, ncols), 1)
    acc = None
    for k, off in enumerate(offs):
        term = (idx[:, k:k + 1] + off == cols).astype(jnp.float32)
        acc = term if acc is None else acc + term
    return acc


def _layer_norm(x, gamma, beta, eps=1e-5):
    m = jnp.mean(x, axis=-1, keepdims=True)
    v = jnp.mean((x - m) ** 2, axis=-1, keepdims=True)
    return (x - m) * jax.lax.rsqrt(v + eps) * gamma + beta


def _fused_kernel(nf_ref, ef_ref, tn_ref, te_ref, wc_ref, bs_ref, be_ref,
                  gn_ref, bn_ref, ge_ref, beta_e_ref,
                  nx_ref, ns_ref, ne_ref, ex_ref):
    n_oh = _onehot_sum(nf_ref[...], _NODE_OFFS, _NODE_COLS)
    nsum = jnp.dot(n_oh, tn_ref[...], preferred_element_type=jnp.float32)
    node_x = _layer_norm(nsum, gn_ref[...], bn_ref[...])
    nx_ref[...] = node_x
    proj = jax.lax.dot_general(
        node_x, wc_ref[...], (((1,), (1,)), ((), ())),
        preferred_element_type=jnp.float32)
    ns_ref[...] = proj[:, :D] + bs_ref[...]
    ne_ref[...] = proj[:, D:] + be_ref[...]
    e_oh = _onehot_sum(ef_ref[...], _EDGE_OFFS, _EDGE_COLS)
    esum = jnp.dot(e_oh, te_ref[...], preferred_element_type=jnp.float32)
    ex_ref[...] = _layer_norm(esum, ge_ref[...], beta_e_ref[...])


@jax.jit
def _run(nf, ef, tn, te, wc, bs, be, gn, bn, ge, bt):
    rows = nf.shape[0]
    grid = (rows // TILE,)

    def row_spec(width):
        return pl.BlockSpec((TILE, width), lambda i: (i, 0))

    def const_spec(a):
        return pl.BlockSpec(a.shape, lambda i: (0,) * a.ndim)

    out_shape = [jax.ShapeDtypeStruct((rows, D), jnp.float32) for _ in range(4)]
    return pl.pallas_call(
        _fused_kernel,
        grid=grid,
        in_specs=[
            row_spec(nf.shape[1]), row_spec(ef.shape[1]),
            const_spec(tn), const_spec(te),
            const_spec(wc), const_spec(bs), const_spec(be),
            const_spec(gn), const_spec(bn), const_spec(ge), const_spec(bt),
        ],
        out_specs=[row_spec(D) for _ in range(4)],
        out_shape=out_shape,
    )(nf, ef, tn, te, wc, bs, be, gn, bn, ge, bt)


def kernel(node_features, edge_features, emb_element, emb_degree, emb_ring,
           emb_hybrid, emb_aromatic, emb_chirality, emb_charge, W_start,
           b_start, W_end, b_end, emb_etype, emb_conj, emb_ering, emb_stereo,
           gamma_node, beta_node, gamma_edge, beta_edge):
    B, N, _ = node_features.shape
    rows = B * N
    nf = node_features.reshape(rows, -1).astype(jnp.int32)
    ef = edge_features.reshape(rows, -1).astype(jnp.int32)
    tn = jnp.concatenate([emb_element, emb_degree, emb_ring, emb_hybrid,
                          emb_aromatic, emb_chirality, emb_charge], axis=0)
    tn = jnp.pad(tn, ((0, _NODE_COLS - _NODE_TOT), (0, 0)))
    te = jnp.concatenate([emb_etype, emb_conj, emb_ering, emb_stereo], axis=0)
    wc = jnp.concatenate([W_start, W_end], axis=0)
    outs = _run(nf, ef, tn, te,
                wc, b_start.reshape(1, D), b_end.reshape(1, D),
                gamma_node.reshape(1, D), beta_node.reshape(1, D),
                gamma_edge.reshape(1, D), beta_edge.reshape(1, D))
    node_x, node_x_start, node_x_end, edge_x = [o.reshape(B, N, D) for o in outs]
    return (node_x, node_x_start, node_x_end, edge_x)
